# baseline (speedup 1.0000x reference)
import functools

import jax
import jax.numpy as jnp
from jax import lax
from jax.experimental import pallas as pl
from jax.experimental.pallas import tpu as pltpu
from jax.experimental.pallas import tpu_sc as plsc

_NC = 2
_NS = 16
_NW = _NC * _NS
_CHUNK = 128


def _sc_gather(uids, iids, user_emb, item_emb):
    b = uids.shape[0]
    d = user_emb.shape[1]
    rows_per_w = b // _NW
    nchunk = rows_per_w // _CHUNK
    mesh = plsc.VectorSubcoreMesh(core_axis_name="c", subcore_axis_name="s")

    @functools.partial(
        pl.kernel,
        out_type=(
            jax.ShapeDtypeStruct((b, d), jnp.float32),
            jax.ShapeDtypeStruct((b, d), jnp.float32),
        ),
        mesh=mesh,
        scratch_types=[
            pltpu.VMEM((rows_per_w,), jnp.int32),
            pltpu.VMEM((rows_per_w,), jnp.int32),
            pltpu.VMEM((rows_per_w, d), jnp.float32),
            pltpu.SemaphoreType.DMA,
            pltpu.SemaphoreType.DMA((2,)),
        ],
    )
    def gather_kernel(u_hbm, i_hbm, ue_hbm, ie_hbm, out_u, out_i,
                      uidx_v, iidx_v, rows_v, sem_g, sem_o):
        wid = lax.axis_index("s") * _NC + lax.axis_index("c")
        base = wid * rows_per_w
        pltpu.sync_copy(u_hbm.at[pl.ds(base, rows_per_w)], uidx_v)
        pltpu.sync_copy(i_hbm.at[pl.ds(base, rows_per_w)], iidx_v)
        half = rows_per_w // 2
        work = []
        for p in range(2):
            work.append((uidx_v, ue_hbm, out_u, p * half))
            work.append((iidx_v, ie_hbm, out_i, p * half))
        outs = [None, None]
        for k, (idx_v, table_hbm, out_hbm, src_off) in enumerate(work):
            slot = k % 2
            buf_off = slot * half
            if outs[slot] is not None:
                outs[slot].wait()
            gathers = [
                pltpu.async_copy(
                    table_hbm.at[idx_v.at[pl.ds(src_off + j * _CHUNK, _CHUNK)]],
                    rows_v.at[pl.ds(buf_off + j * _CHUNK, _CHUNK)],
                    sem_g,
                )
                for j in range(half // _CHUNK)
            ]
            for g in gathers:
                g.wait()
            outs[slot] = pltpu.async_copy(
                rows_v.at[pl.ds(buf_off, half)],
                out_hbm.at[pl.ds(base + src_off, half)],
                sem_o.at[slot],
            )
        for o in outs:
            o.wait()

    return gather_kernel(uids, iids, user_emb, item_emb)


def _mlp_body(u_ref, i_ref, w1_ref, b1_ref, w2_ref, b2_ref,
              w3_ref, b3_ref, o_ref):
    d = u_ref.shape[1]
    ct = (((1,), (1,)), ((), ()))
    h = lax.dot_general(u_ref[...], w1_ref[:, :d], ct,
                        preferred_element_type=jnp.float32)
    h = h + lax.dot_general(i_ref[...], w1_ref[:, d:], ct,
                            preferred_element_type=jnp.float32)
    h = jnp.maximum(h + b1_ref[...], 0.0)
    h2 = lax.dot_general(h, w2_ref[...], ct,
                         preferred_element_type=jnp.float32)
    h2 = jnp.maximum(h2 + b2_ref[...], 0.0)
    z = lax.dot_general(w3_ref[...], h2, ct,
                        preferred_element_type=jnp.float32)
    o_ref[...] = z.reshape(o_ref.shape) + b3_ref[0]


def _tc_mlp(ug, ig, W1, b1, W2, b2, W3, b3, interpret=False):
    b, d = ug.shape
    tile = 2048
    b1r = b1.reshape(1, -1)
    b2r = b2.reshape(1, -1)
    grid = (b // tile,)
    full = lambda shape: pl.BlockSpec(shape, lambda i: (0,) * len(shape))
    return pl.pallas_call(
        _mlp_body,
        grid=grid,
        in_specs=[
            pl.BlockSpec((tile, d), lambda i: (i, 0)),
            pl.BlockSpec((tile, d), lambda i: (i, 0)),
            full(W1.shape),
            full(b1r.shape),
            full(W2.shape),
            full(b2r.shape),
            full(W3.shape),
            pl.BlockSpec(memory_space=pltpu.SMEM),
        ],
        out_specs=pl.BlockSpec((tile,), lambda i: (i,)),
        out_shape=jax.ShapeDtypeStruct((b,), jnp.float32),
        interpret=interpret,
    )(ug, ig, W1, b1r, W2, b2r, W3, b3)


def kernel(user_ids, item_ids, user_emb, item_emb, W1, b1, W2, b2, W3, b3):
    uids = user_ids.astype(jnp.int32)
    iids = item_ids.astype(jnp.int32)
    b = uids.shape[0]
    h = b // 2
    ug0, ig0 = _sc_gather(uids[:h], iids[:h], user_emb, item_emb)
    ug1, ig1 = _sc_gather(uids[h:], iids[h:], user_emb, item_emb)
    o0 = _tc_mlp(ug0, ig0, W1, b1, W2, b2, W3, b3)
    o1 = _tc_mlp(ug1, ig1, W1, b1, W2, b2, W3, b3)
    return jnp.concatenate([o0, o1])


# speedup vs baseline: 1.1138x; 1.1138x over previous
import functools

import jax
import jax.numpy as jnp
from jax import lax
from jax.experimental import pallas as pl
from jax.experimental.pallas import tpu as pltpu
from jax.experimental.pallas import tpu_sc as plsc

_NC = 2
_NS = 16
_NW = _NC * _NS
_CHUNK = 128


def _sc_gather(uids, iids, user_emb, item_emb):
    b = uids.shape[0]
    d = user_emb.shape[1]
    rows_per_w = b // _NW
    nchunk = rows_per_w // _CHUNK
    nslot = min(7, 2 * nchunk)
    mesh = plsc.VectorSubcoreMesh(core_axis_name="c", subcore_axis_name="s")

    @functools.partial(
        pl.kernel,
        out_type=(
            jax.ShapeDtypeStruct((b, d), jnp.float32),
            jax.ShapeDtypeStruct((b, d), jnp.float32),
        ),
        mesh=mesh,
        scratch_types=[
            pltpu.VMEM((rows_per_w,), jnp.int32),
            pltpu.VMEM((rows_per_w,), jnp.int32),
            pltpu.VMEM((nslot * _CHUNK, d), jnp.float32),
            pltpu.SemaphoreType.DMA((nslot,)),
            pltpu.SemaphoreType.DMA((nslot,)),
        ],
    )
    def gather_kernel(u_hbm, i_hbm, ue_hbm, ie_hbm, out_u, out_i,
                      uidx_v, iidx_v, rows_v, sem_g, sem_o):
        wid = lax.axis_index("s") * _NC + lax.axis_index("c")
        base = wid * rows_per_w
        pltpu.sync_copy(u_hbm.at[pl.ds(base, rows_per_w)], uidx_v)
        pltpu.sync_copy(i_hbm.at[pl.ds(base, rows_per_w)], iidx_v)
        work = []
        for j in range(nchunk):
            work.append((uidx_v, ue_hbm, out_u, j * _CHUNK))
            work.append((iidx_v, ie_hbm, out_i, j * _CHUNK))
        nwork = len(work)

        def gather(k):
            idx_v, table_hbm, _, src_off = work[k]
            slot = k % nslot
            return pltpu.async_copy(
                table_hbm.at[idx_v.at[pl.ds(src_off, _CHUNK)]],
                rows_v.at[pl.ds(slot * _CHUNK, _CHUNK)],
                sem_g.at[slot],
            )

        gh = [gather(k) for k in range(min(nslot, nwork))]
        gh += [None] * (nwork - len(gh))
        outs = [None] * nslot
        for k in range(nwork):
            slot = k % nslot
            if gh[k] is None:
                outs[slot].wait()
                gh[k] = gather(k)
            gh[k].wait()
            _, _, out_hbm, src_off = work[k]
            outs[slot] = pltpu.async_copy(
                rows_v.at[pl.ds(slot * _CHUNK, _CHUNK)],
                out_hbm.at[pl.ds(base + src_off, _CHUNK)],
                sem_o.at[slot],
            )
        for o in outs:
            if o is not None:
                o.wait()

    return gather_kernel(uids, iids, user_emb, item_emb)


def _mlp_body(u_ref, i_ref, w1_ref, b1_ref, w2_ref, b2_ref,
              w3_ref, b3_ref, o_ref):
    d = u_ref.shape[1]
    ct = (((1,), (1,)), ((), ()))
    h = lax.dot_general(u_ref[...], w1_ref[:, :d], ct,
                        preferred_element_type=jnp.float32)
    h = h + lax.dot_general(i_ref[...], w1_ref[:, d:], ct,
                            preferred_element_type=jnp.float32)
    h = jnp.maximum(h + b1_ref[...], 0.0)
    h2 = lax.dot_general(h, w2_ref[...], ct,
                         preferred_element_type=jnp.float32)
    h2 = jnp.maximum(h2 + b2_ref[...], 0.0)
    z = lax.dot_general(w3_ref[...], h2, ct,
                        preferred_element_type=jnp.float32)
    o_ref[...] = z.reshape(o_ref.shape) + b3_ref[0]


def _tc_mlp(ug, ig, W1, b1, W2, b2, W3, b3, interpret=False):
    b, d = ug.shape
    tile = 2048
    b1r = b1.reshape(1, -1)
    b2r = b2.reshape(1, -1)
    grid = (b // tile,)
    full = lambda shape: pl.BlockSpec(shape, lambda i: (0,) * len(shape))
    return pl.pallas_call(
        _mlp_body,
        grid=grid,
        in_specs=[
            pl.BlockSpec((tile, d), lambda i: (i, 0)),
            pl.BlockSpec((tile, d), lambda i: (i, 0)),
            full(W1.shape),
            full(b1r.shape),
            full(W2.shape),
            full(b2r.shape),
            full(W3.shape),
            pl.BlockSpec(memory_space=pltpu.SMEM),
        ],
        out_specs=pl.BlockSpec((tile,), lambda i: (i,)),
        out_shape=jax.ShapeDtypeStruct((b,), jnp.float32),
        interpret=interpret,
    )(ug, ig, W1, b1r, W2, b2r, W3, b3)


def kernel(user_ids, item_ids, user_emb, item_emb, W1, b1, W2, b2, W3, b3):
    uids = user_ids.astype(jnp.int32)
    iids = item_ids.astype(jnp.int32)
    ug, ig = _sc_gather(uids, iids, user_emb, item_emb)
    return _tc_mlp(ug, ig, W1, b1, W2, b2, W3, b3)


# speedup vs baseline: 1.1490x; 1.0316x over previous
import functools

import jax
import jax.numpy as jnp
from jax import lax
from jax.experimental import pallas as pl
from jax.experimental.pallas import tpu as pltpu
from jax.experimental.pallas import tpu_sc as plsc

_NC = 2
_NS = 16
_NW = _NC * _NS
_CHUNK = 128


def _sc_gather(uids, iids, user_emb, item_emb):
    b = uids.shape[0]
    d = user_emb.shape[1]
    rows_per_w = b // _NW
    nchunk = rows_per_w // _CHUNK
    nslot = min(7, 2 * nchunk)
    mesh = plsc.VectorSubcoreMesh(core_axis_name="c", subcore_axis_name="s")

    @functools.partial(
        pl.kernel,
        out_type=(
            jax.ShapeDtypeStruct((b, d), jnp.float32),
            jax.ShapeDtypeStruct((b, d), jnp.float32),
        ),
        mesh=mesh,
        scratch_types=[
            pltpu.VMEM((rows_per_w,), jnp.int32),
            pltpu.VMEM((rows_per_w,), jnp.int32),
            pltpu.VMEM((nslot * _CHUNK, d), jnp.float32),
            pltpu.SemaphoreType.DMA((nslot,)),
            pltpu.SemaphoreType.DMA((nslot,)),
        ],
    )
    def gather_kernel(u_hbm, i_hbm, ue_hbm, ie_hbm, out_u, out_i,
                      uidx_v, iidx_v, rows_v, sem_g, sem_o):
        wid = lax.axis_index("s") * _NC + lax.axis_index("c")
        base = wid * rows_per_w
        pltpu.sync_copy(u_hbm.at[pl.ds(base, rows_per_w)], uidx_v)
        pltpu.sync_copy(i_hbm.at[pl.ds(base, rows_per_w)], iidx_v)
        work = []
        for j in range(nchunk):
            work.append((uidx_v, ue_hbm, out_u, j * _CHUNK))
            work.append((iidx_v, ie_hbm, out_i, j * _CHUNK))
        nwork = len(work)

        def gather(k):
            idx_v, table_hbm, _, src_off = work[k]
            slot = k % nslot
            return pltpu.async_copy(
                table_hbm.at[idx_v.at[pl.ds(src_off, _CHUNK)]],
                rows_v.at[pl.ds(slot * _CHUNK, _CHUNK)],
                sem_g.at[slot],
            )

        gh = [gather(k) for k in range(min(nslot, nwork))]
        gh += [None] * (nwork - len(gh))
        outs = [None] * nslot
        for k in range(nwork):
            slot = k % nslot
            if gh[k] is None:
                outs[slot].wait()
                gh[k] = gather(k)
            gh[k].wait()
            _, _, out_hbm, src_off = work[k]
            outs[slot] = pltpu.async_copy(
                rows_v.at[pl.ds(slot * _CHUNK, _CHUNK)],
                out_hbm.at[pl.ds(base + src_off, _CHUNK)],
                sem_o.at[slot],
            )
        for o in outs:
            if o is not None:
                o.wait()

    return gather_kernel(uids, iids, user_emb, item_emb)


def _mlp_body(u_ref, i_ref, w1_ref, b1_ref, w2_ref, b2_ref,
              w3_ref, b3_ref, o_ref):
    d = u_ref.shape[1]
    ct = (((1,), (1,)), ((), ()))
    h = lax.dot_general(u_ref[...], w1_ref[:, :d], ct,
                        preferred_element_type=jnp.float32)
    h = h + lax.dot_general(i_ref[...], w1_ref[:, d:], ct,
                            preferred_element_type=jnp.float32)
    h = jnp.maximum(h + b1_ref[...], 0.0)
    h2 = lax.dot_general(h, w2_ref[...], ct,
                         preferred_element_type=jnp.float32)
    h2 = jnp.maximum(h2 + b2_ref[...], 0.0)
    z = lax.dot_general(w3_ref[...], h2, ct,
                        preferred_element_type=jnp.float32)
    o_ref[...] = z.reshape(o_ref.shape) + b3_ref[0]


def _tc_mlp(ug, ig, W1, b1, W2, b2, W3, b3, interpret=False):
    b, d = ug.shape
    tile = 4096
    b1r = b1.reshape(1, -1)
    b2r = b2.reshape(1, -1)
    grid = (b // tile,)
    full = lambda shape: pl.BlockSpec(shape, lambda i: (0,) * len(shape))
    return pl.pallas_call(
        _mlp_body,
        grid=grid,
        in_specs=[
            pl.BlockSpec((tile, d), lambda i: (i, 0)),
            pl.BlockSpec((tile, d), lambda i: (i, 0)),
            full(W1.shape),
            full(b1r.shape),
            full(W2.shape),
            full(b2r.shape),
            full(W3.shape),
            pl.BlockSpec(memory_space=pltpu.SMEM),
        ],
        out_specs=pl.BlockSpec((tile,), lambda i: (i,)),
        out_shape=jax.ShapeDtypeStruct((b,), jnp.float32),
        interpret=interpret,
    )(ug, ig, W1, b1r, W2, b2r, W3, b3)


def kernel(user_ids, item_ids, user_emb, item_emb, W1, b1, W2, b2, W3, b3):
    uids = user_ids.astype(jnp.int32)
    iids = item_ids.astype(jnp.int32)
    ug, ig = _sc_gather(uids, iids, user_emb, item_emb)
    return _tc_mlp(ug, ig, W1, b1, W2, b2, W3, b3)
